# baseline (speedup 1.0000x reference)
import functools

import jax
import jax.numpy as jnp
from jax import lax
from jax.experimental import pallas as pl
from jax.experimental.pallas import tpu as pltpu

N, D, N_CLS, N_SUP = 2048, 512, 128, 5
BLK = 128
N_BLK = N // BLK
N_QUERY = N - N_CLS * N_SUP


def _body(x_hbm, t_ref, out_ref, xv_ref, wt_ref, wqt_ref, sem):
    f32 = jnp.float32
    bf16 = jnp.bfloat16

    cp = pltpu.make_async_copy(x_hbm, xv_ref, sem)
    cp.start()

    t2d = t_ref[...]

    sub_iota = lax.broadcasted_iota(jnp.int32, (BLK, BLK), 0)
    utri = (sub_iota <= lax.broadcasted_iota(jnp.int32, (BLK, BLK), 1)
            ).astype(bf16)

    carry = jnp.zeros((BLK, 1), f32)
    for b in range(N_BLK):
        ot_b = (t2d[b:b + 1, :] == sub_iota).astype(f32)
        cnt = lax.dot_general(
            ot_b.astype(bf16), utri, (((1,), (0,)), ((), ())),
            preferred_element_type=f32)
        rank = jnp.sum(ot_b * (cnt + carry), axis=0, keepdims=True)
        carry = carry + cnt[:, BLK - 1:BLK]
        s_row = (rank <= float(N_SUP)).astype(f32)
        wt_ref[:, b * BLK:(b + 1) * BLK] = (ot_b * s_row).astype(bf16)
        wqt_ref[:, b * BLK:(b + 1) * BLK] = (ot_b * (1.0 - s_row)).astype(bf16)

    cp.wait()
    x = xv_ref[...]

    x0 = x.astype(bf16)
    xr1 = x - x0.astype(f32)
    x1 = xr1.astype(bf16)
    x2 = (xr1 - x1.astype(f32)).astype(bf16)

    def dotg(a, b, dims):
        return lax.dot_general(a, b, (dims, ((), ())),
                               preferred_element_type=f32)

    wt = wt_ref[...]
    cNL = (((1,), (0,)))
    protos = (dotg(wt, x0, cNL) + dotg(wt, x1, cNL) + dotg(wt, x2, cNL)) \
        * (1.0 / N_SUP)

    psq = lax.dot_general(
        jnp.ones((1, D), f32), protos * protos, (((1,), (1,)), ((), ())),
        preferred_element_type=f32,
        precision=lax.Precision.HIGHEST)

    p0 = protos.astype(bf16)
    pr1 = protos - p0.astype(f32)
    p1 = pr1.astype(bf16)
    p2 = (pr1 - p1.astype(f32)).astype(bf16)
    cD = (((1,), (1,)))
    g = (dotg(x0, p0, cD)
         + (dotg(x0, p1, cD) + dotg(x1, p0, cD))
         + (dotg(x0, p2, cD) + dotg(x1, p1, cD) + dotg(x2, p0, cD)))

    logits = 2.0 * g - psq
    m = jnp.max(logits, axis=1, keepdims=True)
    lse = m + jnp.log(jnp.sum(jnp.exp(logits - m), axis=1, keepdims=True))

    cls_iota = lax.broadcasted_iota(jnp.int32, (N, N_CLS), 1)
    amin = jnp.min(jnp.where(logits == m, cls_iota, N_CLS), axis=1,
                   keepdims=True)
    correct = (amin == cls_iota).astype(bf16)

    wqt = wqt_ref[...]
    eye = (sub_iota == lax.broadcasted_iota(jnp.int32, (BLK, BLK), 1)
           ).astype(f32)

    l0 = logits.astype(bf16)
    l1 = (logits - l0.astype(f32)).astype(bf16)
    cNN = (((1,), (0,)))
    m_tgt = dotg(wqt, l0, cNN) + dotg(wqt, l1, cNN)
    tgt_sum = jnp.sum(m_tgt * eye)

    z0 = lse.astype(bf16)
    z1 = (lse - z0.astype(f32)).astype(bf16)
    qlse = jnp.sum(dotg(wqt, z0, cNN) + dotg(wqt, z1, cNN))

    m_acc = dotg(wqt, correct, cNN)
    acc = jnp.sum(m_acc * eye) * (1.0 / N_QUERY)
    loss = (qlse - tgt_sum) * (1.0 / N_QUERY)

    lane = lax.broadcasted_iota(jnp.int32, (1, N_CLS), 1)
    out_ref[...] = (jnp.where(lane == 0, loss, 0.0)
                    + jnp.where(lane == 1, acc, 0.0))


@functools.partial(jax.jit)
def kernel(input, target):
    t2d = target.reshape(N_BLK, BLK).astype(jnp.int32)
    out = pl.pallas_call(
        _body,
        in_specs=[pl.BlockSpec(memory_space=pltpu.MemorySpace.HBM),
                  pl.BlockSpec(memory_space=pltpu.MemorySpace.VMEM)],
        out_shape=jax.ShapeDtypeStruct((1, N_CLS), jnp.float32),
        scratch_shapes=[pltpu.VMEM((N, D), jnp.float32),
                        pltpu.VMEM((N_CLS, N), jnp.bfloat16),
                        pltpu.VMEM((N_CLS, N), jnp.bfloat16),
                        pltpu.SemaphoreType.DMA],
    )(input, t2d)
    return out[0, 0], out[0, 1]


# speedup vs baseline: 1.1082x; 1.1082x over previous
import functools

import jax
import jax.numpy as jnp
from jax import lax
from jax.experimental import pallas as pl
from jax.experimental.pallas import tpu as pltpu

N, D, N_CLS, N_SUP = 2048, 512, 128, 5
BLK = 128
N_BLK = N // BLK
N_QUERY = N - N_CLS * N_SUP


CHUNK = 512
N_CHUNK = N // CHUNK


def _body(x_hbm, t_ref, loss_ref, acc_ref, xv_ref, wt_ref, wqt_ref, sems):
    f32 = jnp.float32
    bf16 = jnp.bfloat16

    cps = [pltpu.make_async_copy(
        x_hbm.at[pl.ds(k * CHUNK, CHUNK), :],
        xv_ref.at[pl.ds(k * CHUNK, CHUNK), :],
        sems.at[k]) for k in range(N_CHUNK)]
    for cp in cps:
        cp.start()

    t2d = t_ref[...]

    sub_iota = lax.broadcasted_iota(jnp.int32, (BLK, BLK), 0)
    utri = (sub_iota <= lax.broadcasted_iota(jnp.int32, (BLK, BLK), 1)
            ).astype(bf16)

    carry = jnp.zeros((BLK, 1), f32)
    for b in range(N_BLK):
        ot_b = (t2d[b:b + 1, :] == sub_iota).astype(f32)
        cnt = lax.dot_general(
            ot_b.astype(bf16), utri, (((1,), (0,)), ((), ())),
            preferred_element_type=f32)
        rank = jnp.sum(ot_b * (cnt + carry), axis=0, keepdims=True)
        carry = carry + cnt[:, BLK - 1:BLK]
        s_row = (rank <= float(N_SUP)).astype(f32)
        wt_ref[:, b * BLK:(b + 1) * BLK] = (ot_b * s_row).astype(bf16)
        wqt_ref[:, b * BLK:(b + 1) * BLK] = (ot_b * (1.0 - s_row)).astype(bf16)

    x0s, x1s, x2s = [], [], []
    for k in range(N_CHUNK):
        cps[k].wait()
        xc = xv_ref[k * CHUNK:(k + 1) * CHUNK, :]
        c0 = xc.astype(bf16)
        xr1 = xc - c0.astype(f32)
        c1 = xr1.astype(bf16)
        x0s.append(c0)
        x1s.append(c1)
        x2s.append((xr1 - c1.astype(f32)).astype(bf16))
    x0 = jnp.concatenate(x0s, axis=0)
    x1 = jnp.concatenate(x1s, axis=0)
    x2 = jnp.concatenate(x2s, axis=0)

    def dotg(a, b, dims):
        return lax.dot_general(a, b, (dims, ((), ())),
                               preferred_element_type=f32)

    wt = wt_ref[...]
    cNL = (((1,), (0,)))
    protos = (dotg(wt, x0, cNL) + dotg(wt, x1, cNL) + dotg(wt, x2, cNL)) \
        * (1.0 / N_SUP)

    psq = lax.dot_general(
        jnp.ones((1, D), f32), protos * protos, (((1,), (1,)), ((), ())),
        preferred_element_type=f32,
        precision=lax.Precision.HIGHEST)

    p0 = protos.astype(bf16)
    pr1 = protos - p0.astype(f32)
    p1 = pr1.astype(bf16)
    p2 = (pr1 - p1.astype(f32)).astype(bf16)
    cD = (((1,), (1,)))
    g = (dotg(x0, p0, cD)
         + (dotg(x0, p1, cD) + dotg(x1, p0, cD))
         + (dotg(x0, p2, cD) + dotg(x1, p1, cD) + dotg(x2, p0, cD)))

    logits = 2.0 * g - psq
    m = jnp.max(logits, axis=1, keepdims=True)
    lse = m + jnp.log(jnp.sum(jnp.exp(logits - m), axis=1, keepdims=True))

    cls_iota = lax.broadcasted_iota(jnp.int32, (N, N_CLS), 1)
    amin = jnp.min(jnp.where(logits == m, cls_iota, N_CLS), axis=1,
                   keepdims=True)
    correct = (amin == cls_iota).astype(bf16)

    wqt = wqt_ref[...]
    eye = (sub_iota == lax.broadcasted_iota(jnp.int32, (BLK, BLK), 1)
           ).astype(f32)

    l0 = logits.astype(bf16)
    l1 = (logits - l0.astype(f32)).astype(bf16)
    cNN = (((1,), (0,)))
    m_tgt = dotg(wqt, l0, cNN) + dotg(wqt, l1, cNN)
    tgt_sum = jnp.sum(m_tgt * eye)

    z0 = lse.astype(bf16)
    z1 = (lse - z0.astype(f32)).astype(bf16)
    qlse = jnp.sum(dotg(wqt, z0, cNN) + dotg(wqt, z1, cNN))

    m_acc = dotg(wqt, correct, cNN)
    acc = jnp.sum(m_acc * eye) * (1.0 / N_QUERY)
    loss = (qlse - tgt_sum) * (1.0 / N_QUERY)

    loss_ref[0, 0] = loss
    acc_ref[0, 0] = acc


@functools.partial(jax.jit)
def kernel(input, target):
    t2d = target.reshape(N_BLK, BLK).astype(jnp.int32)
    loss, acc = pl.pallas_call(
        _body,
        in_specs=[pl.BlockSpec(memory_space=pltpu.MemorySpace.HBM),
                  pl.BlockSpec(memory_space=pltpu.MemorySpace.VMEM)],
        out_shape=[jax.ShapeDtypeStruct((1, 1), jnp.float32),
                   jax.ShapeDtypeStruct((1, 1), jnp.float32)],
        out_specs=[pl.BlockSpec(memory_space=pltpu.MemorySpace.SMEM),
                   pl.BlockSpec(memory_space=pltpu.MemorySpace.SMEM)],
        scratch_shapes=[pltpu.VMEM((N, D), jnp.float32),
                        pltpu.VMEM((N_CLS, N), jnp.bfloat16),
                        pltpu.VMEM((N_CLS, N), jnp.bfloat16),
                        pltpu.SemaphoreType.DMA((N_CHUNK,))],
    )(input, t2d)
    return loss[0, 0], acc[0, 0]


# speedup vs baseline: 1.1640x; 1.0504x over previous
import functools

import jax
import jax.numpy as jnp
from jax import lax
from jax.experimental import pallas as pl
from jax.experimental.pallas import tpu as pltpu

N, D, N_CLS, N_SUP = 2048, 512, 128, 5
BLK = 128
N_BLK = N // BLK
N_QUERY = N - N_CLS * N_SUP


CHUNK = 512
N_CHUNK = N // CHUNK


def _body(x_hbm, t_ref, loss_ref, acc_ref, xv_ref, wt_ref, wqt_ref, sems):
    f32 = jnp.float32
    bf16 = jnp.bfloat16

    cps = [pltpu.make_async_copy(
        x_hbm.at[pl.ds(k * CHUNK, CHUNK), :],
        xv_ref.at[pl.ds(k * CHUNK, CHUNK), :],
        sems.at[k]) for k in range(N_CHUNK)]
    for cp in cps:
        cp.start()

    t2d = t_ref[...]

    sub_iota = lax.broadcasted_iota(jnp.int32, (BLK, BLK), 0)
    utri = (sub_iota <= lax.broadcasted_iota(jnp.int32, (BLK, BLK), 1)
            ).astype(bf16)

    carry = jnp.zeros((BLK, 1), f32)
    for b in range(N_BLK):
        ot_b = (t2d[b:b + 1, :] == sub_iota).astype(f32)
        cnt = lax.dot_general(
            ot_b.astype(bf16), utri, (((1,), (0,)), ((), ())),
            preferred_element_type=f32)
        rank = jnp.sum(ot_b * (cnt + carry), axis=0, keepdims=True)
        carry = carry + cnt[:, BLK - 1:BLK]
        s_row = (rank <= float(N_SUP)).astype(f32)
        wt_ref[:, b * BLK:(b + 1) * BLK] = (ot_b * s_row).astype(bf16)
        wqt_ref[:, b * BLK:(b + 1) * BLK] = (ot_b * (1.0 - s_row)).astype(bf16)

    def dotg(a, b, dims):
        return lax.dot_general(a, b, (dims, ((), ())),
                               preferred_element_type=f32)

    cNL = (((1,), (0,)))

    x0s, x1s, x2s = [], [], []
    pacc = jnp.zeros((N_CLS, D), f32)
    for k in range(N_CHUNK):
        cps[k].wait()
        xc = xv_ref[k * CHUNK:(k + 1) * CHUNK, :]
        c0 = xc.astype(bf16)
        xr1 = xc - c0.astype(f32)
        c1 = xr1.astype(bf16)
        c2 = (xr1 - c1.astype(f32)).astype(bf16)
        x0s.append(c0)
        x1s.append(c1)
        x2s.append(c2)
        wtk = wt_ref[:, k * CHUNK:(k + 1) * CHUNK]
        pacc = pacc + (dotg(wtk, c0, cNL) + dotg(wtk, c1, cNL)
                       + dotg(wtk, c2, cNL))
    x0 = jnp.concatenate(x0s, axis=0)
    x1 = jnp.concatenate(x1s, axis=0)
    x2 = jnp.concatenate(x2s, axis=0)
    protos = pacc * (1.0 / N_SUP)

    psq = lax.dot_general(
        jnp.ones((1, D), f32), protos * protos, (((1,), (1,)), ((), ())),
        preferred_element_type=f32,
        precision=lax.Precision.HIGHEST)

    p0 = protos.astype(bf16)
    pr1 = protos - p0.astype(f32)
    p1 = pr1.astype(bf16)
    p2 = (pr1 - p1.astype(f32)).astype(bf16)
    cD = (((1,), (1,)))
    g = (dotg(x0, p0, cD)
         + (dotg(x0, p1, cD) + dotg(x1, p0, cD))
         + (dotg(x0, p2, cD) + dotg(x1, p1, cD) + dotg(x2, p0, cD)))

    logits = 2.0 * g - psq
    m = jnp.max(logits, axis=1, keepdims=True)
    lse = m + jnp.log(jnp.sum(jnp.exp(logits - m), axis=1, keepdims=True))

    cls_iota = lax.broadcasted_iota(jnp.int32, (N, N_CLS), 1)
    amin = jnp.min(jnp.where(logits == m, cls_iota, N_CLS), axis=1,
                   keepdims=True)
    correct = (amin == cls_iota).astype(bf16)

    wqt = wqt_ref[...]
    eye = (sub_iota == lax.broadcasted_iota(jnp.int32, (BLK, BLK), 1)
           ).astype(f32)

    l0 = logits.astype(bf16)
    l1 = (logits - l0.astype(f32)).astype(bf16)
    cNN = (((1,), (0,)))
    m_tgt = dotg(wqt, l0, cNN) + dotg(wqt, l1, cNN)
    tgt_sum = jnp.sum(m_tgt * eye)

    z0 = lse.astype(bf16)
    z1 = (lse - z0.astype(f32)).astype(bf16)
    qlse = jnp.sum(dotg(wqt, z0, cNN) + dotg(wqt, z1, cNN))

    m_acc = dotg(wqt, correct, cNN)
    acc = jnp.sum(m_acc * eye) * (1.0 / N_QUERY)
    loss = (qlse - tgt_sum) * (1.0 / N_QUERY)

    loss_ref[0, 0] = loss
    acc_ref[0, 0] = acc


@functools.partial(jax.jit)
def kernel(input, target):
    t2d = target.reshape(N_BLK, BLK).astype(jnp.int32)
    loss, acc = pl.pallas_call(
        _body,
        in_specs=[pl.BlockSpec(memory_space=pltpu.MemorySpace.HBM),
                  pl.BlockSpec(memory_space=pltpu.MemorySpace.VMEM)],
        out_shape=[jax.ShapeDtypeStruct((1, 1), jnp.float32),
                   jax.ShapeDtypeStruct((1, 1), jnp.float32)],
        out_specs=[pl.BlockSpec(memory_space=pltpu.MemorySpace.SMEM),
                   pl.BlockSpec(memory_space=pltpu.MemorySpace.SMEM)],
        scratch_shapes=[pltpu.VMEM((N, D), jnp.float32),
                        pltpu.VMEM((N_CLS, N), jnp.bfloat16),
                        pltpu.VMEM((N_CLS, N), jnp.bfloat16),
                        pltpu.SemaphoreType.DMA((N_CHUNK,))],
    )(input, t2d)
    return loss[0, 0], acc[0, 0]
